# baseline (speedup 1.0000x reference)
import jax
import jax.numpy as jnp
from jax.experimental import pallas as pl
from jax.experimental.pallas import tpu as pltpu

_BLOCK_ROWS = 512


def _zero_fill(o_ref, scratch, sem):
    nblk = o_ref.shape[0] // _BLOCK_ROWS

    def _copy(i):
        src = scratch.at[pl.ds((i % 2) * _BLOCK_ROWS, _BLOCK_ROWS), :]
        dst = o_ref.at[pl.ds(i * _BLOCK_ROWS, _BLOCK_ROWS), :]
        return pltpu.make_async_copy(src, dst, sem.at[i])

    scratch[pl.ds(0, _BLOCK_ROWS), :] = jnp.zeros((_BLOCK_ROWS, o_ref.shape[1]), o_ref.dtype)
    _copy(0).start()
    scratch[pl.ds(_BLOCK_ROWS, _BLOCK_ROWS), :] = jnp.zeros((_BLOCK_ROWS, o_ref.shape[1]), o_ref.dtype)
    for i in range(1, nblk):
        _copy(i).start()
    for i in range(nblk):
        _copy(i).wait()


def kernel(x, W_gate):
    b, s, h = x.shape
    rows = b * s
    out = pl.pallas_call(
        _zero_fill,
        out_specs=pl.BlockSpec(memory_space=pltpu.MemorySpace.HBM),
        out_shape=jax.ShapeDtypeStruct((rows, h), x.dtype),
        scratch_shapes=[
            pltpu.VMEM((2 * _BLOCK_ROWS, h), jnp.float32),
            pltpu.SemaphoreType.DMA((rows // _BLOCK_ROWS,)),
        ],
    )()
    return out.reshape(b, s, h)


# speedup vs baseline: 1.0465x; 1.0465x over previous
import jax
import jax.numpy as jnp
from jax.experimental import pallas as pl
from jax.experimental.pallas import tpu as pltpu


def _zero_block(o_ref):
    o_ref[...] = jnp.zeros_like(o_ref)


def kernel(x, W_gate):
    b, s, h = x.shape
    rows = b * s
    block_rows = 512
    out = pl.pallas_call(
        _zero_block,
        grid=(rows // block_rows,),
        out_specs=pl.BlockSpec((block_rows, h), lambda i: (i, 0)),
        out_shape=jax.ShapeDtypeStruct((rows, h), x.dtype),
        compiler_params=pltpu.CompilerParams(
            dimension_semantics=("parallel",),
        ),
    )()
    return out.reshape(b, s, h)
